# Initial kernel scaffold; baseline (speedup 1.0000x reference)
import jax
import jax.numpy as jnp
from jax.experimental import pallas as pl


def kernel(x, edge_index, We1, be1, We2, be2, We3, be3, Wd1, bd1, Wd2, bd2, Wd3, bd3):
    raise NotImplementedError("write your pallas kernel here")


# speedup vs baseline: 36.7952x; 36.7952x over previous
import functools

import jax
import jax.numpy as jnp
from jax import lax
from jax.experimental import pallas as pl
from jax.experimental.pallas import tpu as pltpu
from jax.experimental.pallas import tpu_sc as plsc

N = 100000
E = 6400000
DP = 16
K = 8
T = 196
NW = 32
RPW = K * T
R = NW * RPW
EPAD = R * 128
NACC = N + 16
NPW = N // 16

def _agg_body(table, src2d, dst2d, zeros_hbm, out, acc, srcv, dstv, rows, gsem):
    c = lax.axis_index("c")
    s = lax.axis_index("s")
    w = s * 2 + c

    pltpu.sync_copy(zeros_hbm, acc.at[pl.ds(s * NPW, NPW)])
    plsc.subcore_barrier()

    base = w * RPW

    def body(o, carry):
        r0 = base + o * K
        pltpu.sync_copy(src2d.at[pl.ds(r0, K)], srcv)
        pltpu.sync_copy(dst2d.at[pl.ds(r0, K)], dstv)
        cps = [pltpu.async_copy(table.at[srcv.at[j]], rows.at[j], gsem)
               for j in range(K)]
        for cp in cps:
            cp.wait()
        for j in range(K):
            pltpu.sync_copy(rows.at[j], acc.at[dstv.at[j]], add=True)
        return carry

    lax.fori_loop(0, T, body, 0)

    plsc.subcore_barrier()
    pltpu.sync_copy(acc.at[pl.ds(s * NPW, NPW)], out.at[c, s])


@functools.cache
def _make_agg():
    return pl.kernel(
        _agg_body,
        out_type=jax.ShapeDtypeStruct((2, 16, NPW, DP), jnp.float32),
        mesh=plsc.VectorSubcoreMesh(core_axis_name="c", subcore_axis_name="s"),
        scratch_types=[
            pltpu.VMEM_SHARED((NACC, DP), jnp.float32),
            pltpu.VMEM((K, 128), jnp.int32),
            pltpu.VMEM((K, 128), jnp.int32),
            pltpu.VMEM((K, 128, DP), jnp.float32),
            pltpu.SemaphoreType.DMA,
        ],
        compiler_params=pltpu.CompilerParams(use_tc_tiling_on_sc=False),
    )


BN = 2000


def _norm_body(p_ref, t_ref):
    a = p_ref[0] + p_ref[1]
    cnt = jnp.maximum(a[:, 10:11], 1.0)
    m = a / cnt
    lane = lax.broadcasted_iota(jnp.int32, (BN, DP), 1)
    t_ref[...] = jnp.where((lane >= 1) & (lane <= 9), m,
                           jnp.where(lane == 10, jnp.float32(1.0),
                                     jnp.float32(0.0)))


_norm = pl.pallas_call(
    _norm_body,
    grid=(N // BN,),
    in_specs=[pl.BlockSpec((2, BN, DP), lambda i: (0, i, 0))],
    out_specs=pl.BlockSpec((BN, DP), lambda i: (i, 0)),
    out_shape=jax.ShapeDtypeStruct((N, DP), jnp.float32),
)


def _mlp_body(x_ref, p_ref, we1a, we1b, be1, we2, be2, we3, be3,
              wd1, bd1, wd2, bd2, wd3, bd3, enc_ref, dec_ref):
    a = p_ref[0] + p_ref[1]
    cnt = jnp.maximum(a[:, 10:11], 1.0)
    agg = a[:, :10] / cnt
    lane = lax.broadcasted_iota(jnp.int32, (BN, 10), 1)
    agg = jnp.where(lane == 0, jnp.float32(0.0), agg)
    xb = x_ref[...]

    def dot(u, wref):
        return jnp.dot(u, wref[...], preferred_element_type=jnp.float32)

    h = jnp.maximum(dot(xb, we1a) + dot(agg, we1b) + be1[...], 0.0)
    h = jnp.maximum(dot(h, we2) + be2[...], 0.0)
    enc = dot(h, we3) + be3[...]
    h = jnp.maximum(dot(enc, wd1) + bd1[...], 0.0)
    h = jnp.maximum(dot(h, wd2) + bd2[...], 0.0)
    dec = dot(h, wd3) + bd3[...]
    enc_ref[...] = enc
    dec_ref[...] = dec


def _wspec(shape):
    nd = len(shape)
    return pl.BlockSpec(shape, lambda i: (0,) * nd)


def _make_mlp():
    wshapes = [(10, 15), (10, 15), (1, 15), (15, 10), (1, 10), (10, 5),
               (1, 5), (5, 10), (1, 10), (10, 15), (1, 15), (15, 20), (1, 20)]
    return pl.pallas_call(
        _mlp_body,
        grid=(N // BN,),
        in_specs=[pl.BlockSpec((BN, 10), lambda i: (i, 0)),
                  pl.BlockSpec((2, BN, DP), lambda i: (0, i, 0))]
                 + [_wspec(s) for s in wshapes],
        out_specs=[pl.BlockSpec((BN, 5), lambda i: (i, 0)),
                   pl.BlockSpec((BN, 20), lambda i: (i, 0))],
        out_shape=[jax.ShapeDtypeStruct((N, 5), jnp.float32),
                   jax.ShapeDtypeStruct((N, 20), jnp.float32)],
    )


_mlp = _make_mlp()


def kernel(x, edge_index, We1, be1, We2, be2, We3, be3,
           Wd1, bd1, Wd2, bd2, Wd3, bd3):
    f32 = jnp.float32
    src = edge_index[0]
    dst = edge_index[1]
    pad = EPAD - E
    srcp = jnp.concatenate([src, jnp.zeros((pad,), jnp.int32)]).reshape(R, 128)
    dstp = jnp.concatenate([dst, jnp.full((pad,), N, jnp.int32)]).reshape(R, 128)
    xp = jnp.concatenate([x, jnp.ones((N, 1), f32), jnp.zeros((N, 5), f32)],
                         axis=1)
    zeros_hbm = jnp.zeros((NPW, DP), f32)

    agg = _make_agg()
    p1 = agg(xp, srcp, dstp, zeros_hbm).reshape(2, N, DP)
    t2 = _norm(p1)
    p2 = agg(t2, srcp, dstp, zeros_hbm).reshape(2, N, DP)
    enc, dec = _mlp(x, p2, We1[:10], We1[10:], be1.reshape(1, -1),
                    We2, be2.reshape(1, -1), We3, be3.reshape(1, -1),
                    Wd1, bd1.reshape(1, -1), Wd2, bd2.reshape(1, -1),
                    Wd3, bd3.reshape(1, -1))
    return enc, dec


# speedup vs baseline: 53.8697x; 1.4640x over previous
import functools

import jax
import jax.numpy as jnp
from jax import lax
from jax.experimental import pallas as pl
from jax.experimental.pallas import tpu as pltpu
from jax.experimental.pallas import tpu_sc as plsc

N = 100000
E = 6400000
DP = 16
K = 6
T = 261
NW = 32
RPW = K * T
R = NW * RPW
EPAD = R * 128
NACC = N + 16
NPW = N // 16

def _agg_body(table, ei2, zeros_hbm, out, acc, sdv, rows, gsem, ssem, isem):
    c = lax.axis_index("c")
    s = lax.axis_index("s")
    w = s * 2 + c

    pltpu.sync_copy(zeros_hbm, acc.at[pl.ds(s * NPW, NPW)])
    plsc.subcore_barrier()

    base = w * RPW

    def idx_fetch(o, slot):
        return pltpu.async_copy(ei2.at[:, pl.ds(base + o * K, K)],
                                sdv.at[slot], isem)

    idx_fetch(0, 0)

    def body(o, carry):
        b = lax.rem(o, 2)
        sl = lax.rem(o, 3)
        pltpu.make_async_copy(ei2.at[:, pl.ds(base, K)],
                              sdv.at[sl], isem).wait()

        @pl.when(o >= 2)
        def _():
            for j in range(K):
                pltpu.make_async_copy(table.at[pl.ds(0, 128)],
                                      rows.at[b, j], ssem).wait()

        @pl.when(o < T - 1)
        def _():
            idx_fetch(o + 1, lax.rem(o + 1, 3))

        gcs = [pltpu.async_copy(table.at[sdv.at[sl, 0, j]], rows.at[b, j],
                                gsem) for j in range(K)]
        for cp in gcs:
            cp.wait()
        for j in range(K):
            pltpu.async_copy(rows.at[b, j], acc.at[sdv.at[sl, 1, j]],
                             ssem, add=True)
        return carry

    lax.fori_loop(0, T, body, 0)

    for j in range(2 * K):
        pltpu.make_async_copy(table.at[pl.ds(0, 128)],
                              rows.at[0, 0], ssem).wait()

    plsc.subcore_barrier()
    pltpu.sync_copy(acc.at[pl.ds(s * NPW, NPW)], out.at[c, s])


@functools.cache
def _make_agg():
    return pl.kernel(
        _agg_body,
        out_type=jax.ShapeDtypeStruct((2, 16, NPW, DP), jnp.float32),
        mesh=plsc.VectorSubcoreMesh(core_axis_name="c", subcore_axis_name="s"),
        scratch_types=[
            pltpu.VMEM_SHARED((NACC, DP), jnp.float32),
            pltpu.VMEM((3, 2, K, 128), jnp.int32),
            pltpu.VMEM((2, K, 128, DP), jnp.float32),
            pltpu.SemaphoreType.DMA,
            pltpu.SemaphoreType.DMA,
            pltpu.SemaphoreType.DMA,
        ],
        compiler_params=pltpu.CompilerParams(use_tc_tiling_on_sc=False),
    )


BN = 2000


def _norm_body(p_ref, t_ref):
    a = p_ref[0] + p_ref[1]
    cnt = jnp.maximum(a[:, 10:11], 1.0)
    m = a / cnt
    lane = lax.broadcasted_iota(jnp.int32, (BN, DP), 1)
    t_ref[...] = jnp.where((lane >= 1) & (lane <= 9), m,
                           jnp.where(lane == 10, jnp.float32(1.0),
                                     jnp.float32(0.0)))


_norm = pl.pallas_call(
    _norm_body,
    grid=(N // BN,),
    in_specs=[pl.BlockSpec((2, BN, DP), lambda i: (0, i, 0))],
    out_specs=pl.BlockSpec((BN, DP), lambda i: (i, 0)),
    out_shape=jax.ShapeDtypeStruct((N, DP), jnp.float32),
)


def _mlp_body(x_ref, p_ref, we1a, we1b, be1, we2, be2, we3, be3,
              wd1, bd1, wd2, bd2, wd3, bd3, enc_ref, dec_ref):
    a = p_ref[0] + p_ref[1]
    cnt = jnp.maximum(a[:, 10:11], 1.0)
    agg = a[:, :10] / cnt
    lane = lax.broadcasted_iota(jnp.int32, (BN, 10), 1)
    agg = jnp.where(lane == 0, jnp.float32(0.0), agg)
    xb = x_ref[...]

    def dot(u, wref):
        return jnp.dot(u, wref[...], preferred_element_type=jnp.float32)

    h = jnp.maximum(dot(xb, we1a) + dot(agg, we1b) + be1[...], 0.0)
    h = jnp.maximum(dot(h, we2) + be2[...], 0.0)
    enc = dot(h, we3) + be3[...]
    h = jnp.maximum(dot(enc, wd1) + bd1[...], 0.0)
    h = jnp.maximum(dot(h, wd2) + bd2[...], 0.0)
    dec = dot(h, wd3) + bd3[...]
    enc_ref[...] = enc
    dec_ref[...] = dec


def _wspec(shape):
    nd = len(shape)
    return pl.BlockSpec(shape, lambda i: (0,) * nd)


def _make_mlp():
    wshapes = [(10, 15), (10, 15), (1, 15), (15, 10), (1, 10), (10, 5),
               (1, 5), (5, 10), (1, 10), (10, 15), (1, 15), (15, 20), (1, 20)]
    return pl.pallas_call(
        _mlp_body,
        grid=(N // BN,),
        in_specs=[pl.BlockSpec((BN, 10), lambda i: (i, 0)),
                  pl.BlockSpec((2, BN, DP), lambda i: (0, i, 0))]
                 + [_wspec(s) for s in wshapes],
        out_specs=[pl.BlockSpec((BN, 5), lambda i: (i, 0)),
                   pl.BlockSpec((BN, 20), lambda i: (i, 0))],
        out_shape=[jax.ShapeDtypeStruct((N, 5), jnp.float32),
                   jax.ShapeDtypeStruct((N, 20), jnp.float32)],
    )


_mlp = _make_mlp()


def kernel(x, edge_index, We1, be1, We2, be2, We3, be3,
           Wd1, bd1, Wd2, bd2, Wd3, bd3):
    f32 = jnp.float32
    pad = EPAD - E
    padblk = jnp.concatenate([jnp.zeros((1, pad), jnp.int32),
                              jnp.full((1, pad), N, jnp.int32)])
    ei2 = jnp.concatenate([edge_index, padblk], axis=1).reshape(2, R, 128)
    xp = jnp.concatenate([x, jnp.ones((N, 1), f32), jnp.zeros((N, 5), f32)],
                         axis=1)
    zeros_hbm = jnp.zeros((NPW, DP), f32)

    agg = _make_agg()
    p1 = agg(xp, ei2, zeros_hbm).reshape(2, N, DP)
    t2 = _norm(p1)
    p2 = agg(t2, ei2, zeros_hbm).reshape(2, N, DP)
    enc, dec = _mlp(x, p2, We1[:10], We1[10:], be1.reshape(1, -1),
                    We2, be2.reshape(1, -1), We3, be3.reshape(1, -1),
                    Wd1, bd1.reshape(1, -1), Wd2, bd2.reshape(1, -1),
                    Wd3, bd3.reshape(1, -1))
    return enc, dec


# speedup vs baseline: 73.1962x; 1.3588x over previous
import functools

import jax
import jax.numpy as jnp
from jax import lax
from jax.experimental import pallas as pl
from jax.experimental.pallas import tpu as pltpu
from jax.experimental.pallas import tpu_sc as plsc

N = 100000
E = 6400000
DP = 16
K = 6
T = 260
NW = 32
RPW = K * T
R = E // 128
REM = R - NW * RPW
NT = 100096
NPW = NT // 16
NPK = NT * DP // 128


def _agg_body(table, ei2, zeros_hbm, out, acc, sdv, rows, gsem, ssem, isem):
    c = lax.axis_index("c")
    s = lax.axis_index("s")
    w = s * 2 + c

    pltpu.sync_copy(zeros_hbm, acc.at[pl.ds(s * NPW, NPW)])
    plsc.subcore_barrier()

    base = w * RPW

    def idx_fetch(o, slot):
        return pltpu.async_copy(ei2.at[:, pl.ds(base + o * K, K)],
                                sdv.at[slot], isem)

    idx_fetch(0, 0)

    def body(o, carry):
        b = lax.rem(o, 2)
        sl = lax.rem(o, 3)
        pltpu.make_async_copy(ei2.at[:, pl.ds(base, K)],
                              sdv.at[sl], isem).wait()

        @pl.when(o >= 2)
        def _():
            for j in range(K):
                pltpu.make_async_copy(table.at[pl.ds(0, 128)],
                                      rows.at[b, j], ssem).wait()

        @pl.when(o < T - 1)
        def _():
            idx_fetch(o + 1, lax.rem(o + 1, 3))

        gcs = [pltpu.async_copy(table.at[sdv.at[sl, 0, j]], rows.at[b, j],
                                gsem) for j in range(K)]
        for cp in gcs:
            cp.wait()
        for j in range(K):
            pltpu.async_copy(rows.at[b, j], acc.at[sdv.at[sl, 1, j]],
                             ssem, add=True)
        return carry

    lax.fori_loop(0, T, body, 0)

    for j in range(2 * K):
        pltpu.make_async_copy(table.at[pl.ds(0, 128)],
                              rows.at[0, 0], ssem).wait()

    nrem = REM // 16

    @pl.when(w < 16)
    def _():
        rb = NW * RPW + w * nrem
        pltpu.sync_copy(ei2.at[:, pl.ds(rb, nrem)],
                        sdv.at[0, :, pl.ds(0, nrem)])
        rcs = [pltpu.async_copy(table.at[sdv.at[0, 0, j]], rows.at[0, j],
                                gsem) for j in range(nrem)]
        for cp in rcs:
            cp.wait()
        for j in range(nrem):
            pltpu.sync_copy(rows.at[0, j], acc.at[sdv.at[0, 1, j]], add=True)

    plsc.subcore_barrier()
    pltpu.sync_copy(acc.at[pl.ds(s * NPW, NPW)], out.at[c, s])


@functools.cache
def _make_agg():
    return pl.kernel(
        _agg_body,
        out_type=jax.ShapeDtypeStruct((2, 16, NPW, DP), jnp.float32),
        mesh=plsc.VectorSubcoreMesh(core_axis_name="c", subcore_axis_name="s"),
        scratch_types=[
            pltpu.VMEM_SHARED((NT, DP), jnp.float32),
            pltpu.VMEM((3, 2, K, 128), jnp.int32),
            pltpu.VMEM((2, K, 128, DP), jnp.float32),
            pltpu.SemaphoreType.DMA,
            pltpu.SemaphoreType.DMA,
            pltpu.SemaphoreType.DMA,
        ],
        compiler_params=pltpu.CompilerParams(use_tc_tiling_on_sc=False),
    )


GRID = 4
BR = NPK // GRID


def _norm_body(p_ref, psel_ref, t_ref):
    a = p_ref[0] + p_ref[1]
    cnt = jnp.maximum(jnp.dot(a, psel_ref[...],
                              preferred_element_type=jnp.float32), 1.0)
    m = a / cnt
    lane = lax.rem(lax.broadcasted_iota(jnp.int32, (BR, 128), 1), 16)
    t_ref[...] = jnp.where((lane >= 1) & (lane <= 9), m,
                           jnp.where(lane == 10, jnp.float32(1.0),
                                     jnp.float32(0.0)))


_norm = pl.pallas_call(
    _norm_body,
    grid=(GRID,),
    in_specs=[pl.BlockSpec((2, BR, 128), lambda i: (0, i, 0)),
              pl.BlockSpec((128, 128), lambda i: (0, 0))],
    out_specs=pl.BlockSpec((BR, 128), lambda i: (i, 0)),
    out_shape=jax.ShapeDtypeStruct((NPK, 128), jnp.float32),
)


def _mlp_body(x_ref, p_ref, psel_ref, w1x, w1a, b1, w2, b2, w3, b3,
              wd1, bd1, wd2, bd2, wd3, bd3, enc_ref, dec_ref):
    a = p_ref[0] + p_ref[1]
    cnt = jnp.maximum(jnp.dot(a, psel_ref[...],
                              preferred_element_type=jnp.float32), 1.0)
    an = a / cnt

    def dot(u, wref):
        return jnp.dot(u, wref[...], preferred_element_type=jnp.float32)

    h = jnp.maximum(dot(x_ref[...], w1x) + dot(an, w1a) + b1[...], 0.0)
    h = jnp.maximum(dot(h, w2) + b2[...], 0.0)
    enc = dot(h, w3) + b3[...]
    h = jnp.maximum(dot(enc, wd1) + bd1[...], 0.0)
    h = jnp.maximum(dot(h, wd2) + bd2[...], 0.0)
    dec_ref[...] = dot(h, wd3) + bd3[...]
    enc_ref[...] = enc


def _wspec(shape):
    nd = len(shape)
    return pl.BlockSpec(shape, lambda i: (0,) * nd)


_MLP_WSHAPES = [(128, 128), (80, 120), (128, 120), (1, 120), (120, 80),
                (1, 80), (80, 40), (1, 40), (40, 80), (1, 80), (80, 120),
                (1, 120), (120, 160), (1, 160)]

_mlp = pl.pallas_call(
    _mlp_body,
    grid=(GRID,),
    in_specs=[pl.BlockSpec((BR, 80), lambda i: (i, 0)),
              pl.BlockSpec((2, BR, 128), lambda i: (0, i, 0))]
             + [_wspec(s) for s in _MLP_WSHAPES],
    out_specs=[pl.BlockSpec((BR, 40), lambda i: (i, 0)),
               pl.BlockSpec((BR, 160), lambda i: (i, 0))],
    out_shape=[jax.ShapeDtypeStruct((NPK, 40), jnp.float32),
               jax.ShapeDtypeStruct((NPK, 160), jnp.float32)],
)


def kernel(x, edge_index, We1, be1, We2, be2, We3, be3,
           Wd1, bd1, Wd2, bd2, Wd3, bd3):
    f32 = jnp.float32
    ei2 = edge_index.reshape(2, R, 128)

    xp = jnp.concatenate(
        [x, jnp.ones((N, 1), f32), jnp.zeros((N, 5), f32)], axis=1)
    xp = jnp.concatenate([xp, jnp.zeros((NT - N, DP), f32)])
    zeros_hbm = jnp.zeros((NPW, DP), f32)

    eye8 = jnp.eye(8, dtype=f32)
    psel = jnp.kron(eye8, jnp.zeros((16, 16), f32).at[10, :].set(1.0))
    w16 = jnp.zeros((16, 15), f32).at[1:10, :].set(We1[10:][1:10, :])
    w1x = jnp.kron(eye8, We1[:10])
    w1a = jnp.kron(eye8, w16)

    def pk(w):
        return jnp.kron(eye8, w)

    def bk(b):
        return jnp.tile(b, 8).reshape(1, -1)

    agg = _make_agg()
    p1 = agg(xp, ei2, zeros_hbm).reshape(2, NPK, 128)
    t2 = _norm(p1, psel)
    p2 = agg(t2.reshape(NT, DP), ei2, zeros_hbm).reshape(2, NPK, 128)

    xpk = jnp.concatenate([x, jnp.zeros((NT - N, 10), f32)]).reshape(NPK, 80)
    enc_pk, dec_pk = _mlp(
        xpk, p2, psel, w1x, w1a, bk(be1), pk(We2), bk(be2), pk(We3), bk(be3),
        pk(Wd1), bk(bd1), pk(Wd2), bk(bd2), pk(Wd3), bk(bd3))
    enc = enc_pk.reshape(NT, 5)[:N]
    dec = dec_pk.reshape(NT, 20)[:N]
    return enc, dec
